# baseline (speedup 1.0000x reference)
import functools

import jax
import jax.numpy as jnp
from jax import lax
from jax.experimental import pallas as pl
from jax.experimental.pallas import tpu as pltpu
from jax.experimental.pallas import tpu_sc as plsc

_C = 128
_BE = 2000
_BN = 1000
_NW = 32
_ZR = 125


def _sc_gather(h, coordp, row2d, col2d):
    n, d = h.shape
    nchunk = row2d.shape[0]
    e = nchunk * _C
    iters = (nchunk + _NW - 1) // _NW
    mesh = plsc.VectorSubcoreMesh(core_axis_name="c", subcore_axis_name="s")
    f32 = jnp.float32
    bf16 = jnp.bfloat16

    @functools.partial(
        pl.kernel,
        out_type=(
            jax.ShapeDtypeStruct((e, d), bf16),
            jax.ShapeDtypeStruct((e, d), bf16),
            jax.ShapeDtypeStruct((e, 16), f32),
            jax.ShapeDtypeStruct((e, 16), f32),
        ),
        mesh=mesh,
        scratch_types=[
            pltpu.VMEM((_C,), jnp.int32),
            pltpu.VMEM((_C,), jnp.int32),
            pltpu.VMEM((_C, d), bf16),
            pltpu.VMEM((_C, d), bf16),
            pltpu.VMEM((_C, 16), f32),
            pltpu.VMEM((_C, 16), f32),
            pltpu.SemaphoreType.DMA,
            pltpu.SemaphoreType.DMA,
            pltpu.SemaphoreType.DMA,
            pltpu.SemaphoreType.DMA,
        ],
        compiler_params=pltpu.CompilerParams(use_tc_tiling_on_sc=False),
    )
    def gk(h_hbm, cp_hbm, r2_hbm, c2_hbm, hr_o, hc_o, cr_o, cc_o,
           ir, ic, bhr, bhc, bcr, bcc, s1, s2, s3, s4):
        wid = lax.axis_index("s") * 2 + lax.axis_index("c")

        def body(i, carry):
            ch = wid + i * _NW

            @pl.when(ch < nchunk)
            def _():
                pltpu.sync_copy(r2_hbm.at[ch], ir)
                pltpu.sync_copy(c2_hbm.at[ch], ic)
                d1 = pltpu.async_copy(h_hbm.at[ir], bhr, s1)
                d2 = pltpu.async_copy(h_hbm.at[ic], bhc, s2)
                d3 = pltpu.async_copy(cp_hbm.at[ir], bcr, s3)
                d4 = pltpu.async_copy(cp_hbm.at[ic], bcc, s4)
                d1.wait()
                d2.wait()
                d3.wait()
                d4.wait()
                base = ch * _C
                pltpu.sync_copy(bhr, hr_o.at[pl.ds(base, _C)])
                pltpu.sync_copy(bhc, hc_o.at[pl.ds(base, _C)])
                pltpu.sync_copy(bcr, cr_o.at[pl.ds(base, _C)])
                pltpu.sync_copy(bcc, cc_o.at[pl.ds(base, _C)])

            return carry

        lax.fori_loop(0, iters, body, 0)

    return gk(h, coordp, row2d, col2d)


def _sc_scatter(ef, tc, row2d, n):
    e, d = ef.shape
    nchunk = row2d.shape[0]
    iters = (nchunk + _NW - 1) // _NW
    npt = n // 16
    mesh = plsc.VectorSubcoreMesh(core_axis_name="c", subcore_axis_name="s")
    f32 = jnp.float32

    @functools.partial(
        pl.kernel,
        out_type=(
            jax.ShapeDtypeStruct((n, d), f32),
            jax.ShapeDtypeStruct((n, d), f32),
            jax.ShapeDtypeStruct((n, 16), f32),
            jax.ShapeDtypeStruct((n, 16), f32),
        ),
        mesh=mesh,
        scratch_types=[
            pltpu.VMEM((_C,), jnp.int32),
            pltpu.VMEM((_C, d), f32),
            pltpu.VMEM((_C, 16), f32),
            pltpu.VMEM((_ZR, d), f32),
            pltpu.VMEM((_ZR, 16), f32),
            pltpu.VMEM_SHARED((n, d), f32),
            pltpu.VMEM_SHARED((n, 16), f32),
        ],
        compiler_params=pltpu.CompilerParams(use_tc_tiling_on_sc=False),
    )
    def sk(ef_hbm, tc_hbm, r2_hbm, an0, an1, at0, at1,
           idx, bef, btc, zb1, zb2, accn, acct):
        c = lax.axis_index("c")
        s = lax.axis_index("s")
        wid = s * 2 + c

        def zrow(i, carry):
            for j in range(d // 16):
                zb1[i, pl.ds(j * 16, 16)] = jnp.zeros((16,), f32)
            zb2[i, pl.ds(0, 16)] = jnp.zeros((16,), f32)
            return carry

        lax.fori_loop(0, _ZR, zrow, 0)
        for k in range(npt // _ZR):
            pltpu.sync_copy(zb1, accn.at[pl.ds(s * npt + k * _ZR, _ZR)])
            pltpu.sync_copy(zb2, acct.at[pl.ds(s * npt + k * _ZR, _ZR)])
        plsc.subcore_barrier()

        def body(i, carry):
            ch = wid + i * _NW

            @pl.when(ch < nchunk)
            def _():
                pltpu.sync_copy(r2_hbm.at[ch], idx)
                base = ch * _C
                pltpu.sync_copy(ef_hbm.at[pl.ds(base, _C)], bef)
                pltpu.sync_copy(tc_hbm.at[pl.ds(base, _C)], btc)
                pltpu.sync_copy(bef, accn.at[idx], add=True)
                pltpu.sync_copy(btc, acct.at[idx], add=True)

            return carry

        lax.fori_loop(0, iters, body, 0)
        plsc.subcore_barrier()

        @pl.when(c == 0)
        def _():
            pltpu.sync_copy(accn.at[pl.ds(s * npt, npt)], an0.at[pl.ds(s * npt, npt)])
            pltpu.sync_copy(acct.at[pl.ds(s * npt, npt)], at0.at[pl.ds(s * npt, npt)])

        @pl.when(c == 1)
        def _():
            pltpu.sync_copy(accn.at[pl.ds(s * npt, npt)], an1.at[pl.ds(s * npt, npt)])
            pltpu.sync_copy(acct.at[pl.ds(s * npt, npt)], at1.at[pl.ds(s * npt, npt)])

    return sk(ef, tc, row2d)


def _tc_edge(hr, hc, ea, cr, cc, W1h, W1c, w1r, W1a, b1, W2, b2, Wc1, bc1, wc2):
    e, d = hr.shape
    he = W2.shape[1]
    f32 = jnp.float32

    bf16 = jnp.bfloat16

    def body(hr_r, hc_r, ea_r, cr_r, cc_r, W1h_r, W1c_r, w1r_r, W1a_r, b1_r,
             W2_r, b2_r, Wc1_r, bc1_r, wc2_r, ef_o, tc_o):
        dif = cr_r[...] - cc_r[...]
        radial = jnp.sum(dif * dif, axis=1, keepdims=True)
        x = (jnp.dot(hr_r[...], W1h_r[...], preferred_element_type=f32)
             + jnp.dot(hc_r[...], W1c_r[...], preferred_element_type=f32)
             + jnp.dot(ea_r[...].astype(bf16), W1a_r[...],
                       preferred_element_type=f32)
             + radial * w1r_r[...]
             + b1_r[...])
        x = jnp.maximum(x, 0.0).astype(bf16)
        ef = jnp.maximum(jnp.dot(x, W2_r[...], preferred_element_type=f32)
                         + b2_r[...], 0.0)
        c1 = jnp.maximum(jnp.dot(ef.astype(bf16), Wc1_r[...],
                                 preferred_element_type=f32)
                         + bc1_r[...], 0.0)
        scal = jnp.sum(c1 * wc2_r[...], axis=1, keepdims=True)
        tr = jnp.clip(dif * scal, -100.0, 100.0)
        lane = lax.broadcasted_iota(jnp.int32, (_BE, 16), 1)
        tc_o[...] = tr + jnp.where(lane == 3, 1.0, 0.0)
        ef_o[...] = ef

    wspec = pl.BlockSpec((d, he), lambda i: (0, 0))
    vspec = pl.BlockSpec((1, he), lambda i: (0, 0))
    return pl.pallas_call(
        body,
        grid=(e // _BE,),
        in_specs=[
            pl.BlockSpec((_BE, d), lambda i: (i, 0)),
            pl.BlockSpec((_BE, d), lambda i: (i, 0)),
            pl.BlockSpec((_BE, d), lambda i: (i, 0)),
            pl.BlockSpec((_BE, 16), lambda i: (i, 0)),
            pl.BlockSpec((_BE, 16), lambda i: (i, 0)),
            wspec, wspec, vspec, wspec, vspec,
            wspec, vspec, wspec, vspec, vspec,
        ],
        out_specs=[
            pl.BlockSpec((_BE, he), lambda i: (i, 0)),
            pl.BlockSpec((_BE, 16), lambda i: (i, 0)),
        ],
        out_shape=[
            jax.ShapeDtypeStruct((e, he), f32),
            jax.ShapeDtypeStruct((e, 16), f32),
        ],
    )(hr, hc, ea, cr, cc, W1h, W1c, w1r, W1a, b1, W2, b2, Wc1, bc1, wc2)


def _tc_node(h, coordp, an0, an1, at0, at1, Wn1h, Wn1a, bn1, Wn2, bn2):
    n, d = h.shape
    f32 = jnp.float32

    def body(h_r, cp_r, an0_r, an1_r, at0_r, at1_r, Wn1h_r, Wn1a_r, bn1_r,
             Wn2_r, bn2_r, ho_o, co_o):
        aggn = an0_r[...] + an1_r[...]
        aggt = at0_r[...] + at1_r[...]
        hid = jnp.maximum(
            jnp.dot(h_r[...], Wn1h_r[...], preferred_element_type=f32)
            + jnp.dot(aggn, Wn1a_r[...], preferred_element_type=f32)
            + bn1_r[...], 0.0)
        ho_o[...] = (jnp.dot(hid, Wn2_r[...], preferred_element_type=f32)
                     + bn2_r[...] + h_r[...])
        lane = lax.broadcasted_iota(jnp.int32, (_BN, 16), 1)
        cnt = jnp.sum(jnp.where(lane == 3, aggt, 0.0), axis=1, keepdims=True)
        cnt = jnp.maximum(cnt, 1.0)
        co_o[...] = cp_r[...] + jnp.where(lane < 3, aggt / cnt, 0.0)

    wspec = pl.BlockSpec((d, d), lambda i: (0, 0))
    vspec = pl.BlockSpec((1, d), lambda i: (0, 0))
    return pl.pallas_call(
        body,
        grid=(n // _BN,),
        in_specs=[
            pl.BlockSpec((_BN, d), lambda i: (i, 0)),
            pl.BlockSpec((_BN, 16), lambda i: (i, 0)),
            pl.BlockSpec((_BN, d), lambda i: (i, 0)),
            pl.BlockSpec((_BN, d), lambda i: (i, 0)),
            pl.BlockSpec((_BN, 16), lambda i: (i, 0)),
            pl.BlockSpec((_BN, 16), lambda i: (i, 0)),
            wspec, wspec, vspec, wspec, vspec,
        ],
        out_specs=[
            pl.BlockSpec((_BN, d), lambda i: (i, 0)),
            pl.BlockSpec((_BN, 16), lambda i: (i, 0)),
        ],
        out_shape=[
            jax.ShapeDtypeStruct((n, d), f32),
            jax.ShapeDtypeStruct((n, 16), f32),
        ],
    )(h, coordp, an0, an1, at0, at1, Wn1h, Wn1a, bn1, Wn2, bn2)


def kernel(h, edge_index, coord, edge_attr, W_e1, b_e1, W_e2, b_e2,
           W_n1, b_n1, W_n2, b_n2, W_c1, b_c1, W_c2):
    n, d = h.shape
    e = edge_index.shape[1]
    f32 = jnp.float32

    row2d = edge_index[0].reshape(e // _C, _C)
    col2d = edge_index[1].reshape(e // _C, _C)
    coordp = jnp.concatenate(
        [coord, jnp.zeros((n, 13), f32)], axis=1)

    bf16 = jnp.bfloat16
    hr, hc, cr, cc = _sc_gather(h.astype(bf16), coordp, row2d, col2d)

    W1h = W_e1[:d].astype(bf16)
    W1c = W_e1[d:2 * d].astype(bf16)
    w1r = W_e1[2 * d:2 * d + 1]
    W1a = W_e1[2 * d + 1:].astype(bf16)
    ef, tc = _tc_edge(hr, hc, edge_attr, cr, cc,
                      W1h, W1c, w1r, W1a, b_e1.reshape(1, -1),
                      W_e2.astype(bf16), b_e2.reshape(1, -1),
                      W_c1.astype(bf16), b_c1.reshape(1, -1),
                      W_c2.reshape(1, -1))

    an0, an1, at0, at1 = _sc_scatter(ef, tc, row2d, n)

    h_out, co = _tc_node(h, coordp, an0, an1, at0, at1,
                         W_n1[:d], W_n1[d:], b_n1.reshape(1, -1),
                         W_n2, b_n2.reshape(1, -1))
    coord_out = co[:, :3].reshape(n, 3, 1)
    return (h_out, coord_out, edge_attr)


# speedup vs baseline: 1.3635x; 1.3635x over previous
import functools

import jax
import jax.numpy as jnp
from jax import lax
from jax.experimental import pallas as pl
from jax.experimental.pallas import tpu as pltpu
from jax.experimental.pallas import tpu_sc as plsc

_C = 128
_BE = 2000
_BN = 1000
_NW = 32
_ZR = 125


def _sc_gather(h, coordp, row2d, col2d):
    n, d = h.shape
    nchunk = row2d.shape[0]
    e = nchunk * _C
    iters = (nchunk + _NW - 1) // _NW
    mesh = plsc.VectorSubcoreMesh(core_axis_name="c", subcore_axis_name="s")
    f32 = jnp.float32
    bf16 = jnp.bfloat16

    @functools.partial(
        pl.kernel,
        out_type=(
            jax.ShapeDtypeStruct((e, d), f32),
            jax.ShapeDtypeStruct((e, d), f32),
            jax.ShapeDtypeStruct((e, 16), f32),
            jax.ShapeDtypeStruct((e, 16), f32),
        ),
        mesh=mesh,
        scratch_types=[
            pltpu.VMEM((_C,), jnp.int32),
            pltpu.VMEM((_C,), jnp.int32),
            pltpu.VMEM((_C, d), f32),
            pltpu.VMEM((_C, d), f32),
            pltpu.VMEM((_C, 16), f32),
            pltpu.VMEM((_C, 16), f32),
            pltpu.SemaphoreType.DMA,
            pltpu.SemaphoreType.DMA,
            pltpu.SemaphoreType.DMA,
            pltpu.SemaphoreType.DMA,
        ],
        compiler_params=pltpu.CompilerParams(use_tc_tiling_on_sc=False),
    )
    def gk(h_hbm, cp_hbm, r2_hbm, c2_hbm, hr_o, hc_o, cr_o, cc_o,
           ir, ic, bhr, bhc, bcr, bcc, s1, s2, s3, s4):
        wid = lax.axis_index("s") * 2 + lax.axis_index("c")

        def body(i, carry):
            ch = wid + i * _NW

            @pl.when(ch < nchunk)
            def _():
                pltpu.sync_copy(r2_hbm.at[ch], ir)
                pltpu.sync_copy(c2_hbm.at[ch], ic)
                d1 = pltpu.async_copy(h_hbm.at[ir], bhr, s1)
                d2 = pltpu.async_copy(h_hbm.at[ic], bhc, s2)
                d3 = pltpu.async_copy(cp_hbm.at[ir], bcr, s3)
                d4 = pltpu.async_copy(cp_hbm.at[ic], bcc, s4)
                d1.wait()
                d2.wait()
                d3.wait()
                d4.wait()
                base = ch * _C
                pltpu.sync_copy(bhr, hr_o.at[pl.ds(base, _C)])
                pltpu.sync_copy(bhc, hc_o.at[pl.ds(base, _C)])
                pltpu.sync_copy(bcr, cr_o.at[pl.ds(base, _C)])
                pltpu.sync_copy(bcc, cc_o.at[pl.ds(base, _C)])

            return carry

        lax.fori_loop(0, iters, body, 0)

    return gk(h, coordp, row2d, col2d)


def _sc_scatter(ef, tc, row2d, n):
    e, d = ef.shape
    nchunk = row2d.shape[0]
    iters = (nchunk + _NW - 1) // _NW
    npt = n // 16
    mesh = plsc.VectorSubcoreMesh(core_axis_name="c", subcore_axis_name="s")
    f32 = jnp.float32

    @functools.partial(
        pl.kernel,
        out_type=(
            jax.ShapeDtypeStruct((n, d), f32),
            jax.ShapeDtypeStruct((n, d), f32),
            jax.ShapeDtypeStruct((n, 16), f32),
            jax.ShapeDtypeStruct((n, 16), f32),
        ),
        mesh=mesh,
        scratch_types=[
            pltpu.VMEM((_C,), jnp.int32),
            pltpu.VMEM((_C, d), f32),
            pltpu.VMEM((_C, 16), f32),
            pltpu.VMEM((_ZR, d), f32),
            pltpu.VMEM((_ZR, 16), f32),
            pltpu.VMEM_SHARED((n, d), f32),
            pltpu.VMEM_SHARED((n, 16), f32),
        ],
        compiler_params=pltpu.CompilerParams(use_tc_tiling_on_sc=False),
    )
    def sk(ef_hbm, tc_hbm, r2_hbm, an0, an1, at0, at1,
           idx, bef, btc, zb1, zb2, accn, acct):
        c = lax.axis_index("c")
        s = lax.axis_index("s")
        wid = s * 2 + c

        def zrow(i, carry):
            for j in range(d // 16):
                zb1[i, pl.ds(j * 16, 16)] = jnp.zeros((16,), f32)
            zb2[i, pl.ds(0, 16)] = jnp.zeros((16,), f32)
            return carry

        lax.fori_loop(0, _ZR, zrow, 0)
        for k in range(npt // _ZR):
            pltpu.sync_copy(zb1, accn.at[pl.ds(s * npt + k * _ZR, _ZR)])
            pltpu.sync_copy(zb2, acct.at[pl.ds(s * npt + k * _ZR, _ZR)])
        plsc.subcore_barrier()

        def body(i, carry):
            ch = wid + i * _NW

            @pl.when(ch < nchunk)
            def _():
                pltpu.sync_copy(r2_hbm.at[ch], idx)
                base = ch * _C
                pltpu.sync_copy(ef_hbm.at[pl.ds(base, _C)], bef)
                pltpu.sync_copy(tc_hbm.at[pl.ds(base, _C)], btc)
                pltpu.sync_copy(bef, accn.at[idx], add=True)
                pltpu.sync_copy(btc, acct.at[idx], add=True)

            return carry

        lax.fori_loop(0, iters, body, 0)
        plsc.subcore_barrier()

        @pl.when(c == 0)
        def _():
            pltpu.sync_copy(accn.at[pl.ds(s * npt, npt)], an0.at[pl.ds(s * npt, npt)])
            pltpu.sync_copy(acct.at[pl.ds(s * npt, npt)], at0.at[pl.ds(s * npt, npt)])

        @pl.when(c == 1)
        def _():
            pltpu.sync_copy(accn.at[pl.ds(s * npt, npt)], an1.at[pl.ds(s * npt, npt)])
            pltpu.sync_copy(acct.at[pl.ds(s * npt, npt)], at1.at[pl.ds(s * npt, npt)])

    return sk(ef, tc, row2d)


def _tc_edge(hr, hc, ea, cr, cc, W1h, W1c, w1r, W1a, b1, W2, b2, Wc1, bc1, wc2):
    e, d = hr.shape
    he = W2.shape[1]
    f32 = jnp.float32

    bf16 = jnp.bfloat16

    def body(hr_r, hc_r, ea_r, cr_r, cc_r, W1h_r, W1c_r, w1r_r, W1a_r, b1_r,
             W2_r, b2_r, Wc1_r, bc1_r, wc2_r, ef_o, tc_o):
        dif = cr_r[...] - cc_r[...]
        radial = jnp.sum(dif * dif, axis=1, keepdims=True)
        x = (jnp.dot(hr_r[...].astype(bf16), W1h_r[...],
                     preferred_element_type=f32)
             + jnp.dot(hc_r[...].astype(bf16), W1c_r[...],
                       preferred_element_type=f32)
             + jnp.dot(ea_r[...].astype(bf16), W1a_r[...],
                       preferred_element_type=f32)
             + radial * w1r_r[...]
             + b1_r[...])
        x = jnp.maximum(x, 0.0).astype(bf16)
        ef = jnp.maximum(jnp.dot(x, W2_r[...], preferred_element_type=f32)
                         + b2_r[...], 0.0)
        c1 = jnp.maximum(jnp.dot(ef.astype(bf16), Wc1_r[...],
                                 preferred_element_type=f32)
                         + bc1_r[...], 0.0)
        scal = jnp.sum(c1 * wc2_r[...], axis=1, keepdims=True)
        tr = jnp.clip(dif * scal, -100.0, 100.0)
        lane = lax.broadcasted_iota(jnp.int32, (_BE, 16), 1)
        tc_o[...] = tr + jnp.where(lane == 3, 1.0, 0.0)
        ef_o[...] = ef

    wspec = pl.BlockSpec((d, he), lambda i: (0, 0))
    vspec = pl.BlockSpec((1, he), lambda i: (0, 0))
    return pl.pallas_call(
        body,
        grid=(e // _BE,),
        in_specs=[
            pl.BlockSpec((_BE, d), lambda i: (i, 0)),
            pl.BlockSpec((_BE, d), lambda i: (i, 0)),
            pl.BlockSpec((_BE, d), lambda i: (i, 0)),
            pl.BlockSpec((_BE, 16), lambda i: (i, 0)),
            pl.BlockSpec((_BE, 16), lambda i: (i, 0)),
            wspec, wspec, vspec, wspec, vspec,
            wspec, vspec, wspec, vspec, vspec,
        ],
        out_specs=[
            pl.BlockSpec((_BE, he), lambda i: (i, 0)),
            pl.BlockSpec((_BE, 16), lambda i: (i, 0)),
        ],
        out_shape=[
            jax.ShapeDtypeStruct((e, he), f32),
            jax.ShapeDtypeStruct((e, 16), f32),
        ],
    )(hr, hc, ea, cr, cc, W1h, W1c, w1r, W1a, b1, W2, b2, Wc1, bc1, wc2)


def _tc_node(h, coordp, an0, an1, at0, at1, Wn1h, Wn1a, bn1, Wn2, bn2):
    n, d = h.shape
    f32 = jnp.float32

    def body(h_r, cp_r, an0_r, an1_r, at0_r, at1_r, Wn1h_r, Wn1a_r, bn1_r,
             Wn2_r, bn2_r, ho_o, co_o):
        aggn = an0_r[...] + an1_r[...]
        aggt = at0_r[...] + at1_r[...]
        hid = jnp.maximum(
            jnp.dot(h_r[...], Wn1h_r[...], preferred_element_type=f32)
            + jnp.dot(aggn, Wn1a_r[...], preferred_element_type=f32)
            + bn1_r[...], 0.0)
        ho_o[...] = (jnp.dot(hid, Wn2_r[...], preferred_element_type=f32)
                     + bn2_r[...] + h_r[...])
        lane = lax.broadcasted_iota(jnp.int32, (_BN, 16), 1)
        cnt = jnp.sum(jnp.where(lane == 3, aggt, 0.0), axis=1, keepdims=True)
        cnt = jnp.maximum(cnt, 1.0)
        co_o[...] = cp_r[...] + jnp.where(lane < 3, aggt / cnt, 0.0)

    wspec = pl.BlockSpec((d, d), lambda i: (0, 0))
    vspec = pl.BlockSpec((1, d), lambda i: (0, 0))
    return pl.pallas_call(
        body,
        grid=(n // _BN,),
        in_specs=[
            pl.BlockSpec((_BN, d), lambda i: (i, 0)),
            pl.BlockSpec((_BN, 16), lambda i: (i, 0)),
            pl.BlockSpec((_BN, d), lambda i: (i, 0)),
            pl.BlockSpec((_BN, d), lambda i: (i, 0)),
            pl.BlockSpec((_BN, 16), lambda i: (i, 0)),
            pl.BlockSpec((_BN, 16), lambda i: (i, 0)),
            wspec, wspec, vspec, wspec, vspec,
        ],
        out_specs=[
            pl.BlockSpec((_BN, d), lambda i: (i, 0)),
            pl.BlockSpec((_BN, 16), lambda i: (i, 0)),
        ],
        out_shape=[
            jax.ShapeDtypeStruct((n, d), f32),
            jax.ShapeDtypeStruct((n, 16), f32),
        ],
    )(h, coordp, an0, an1, at0, at1, Wn1h, Wn1a, bn1, Wn2, bn2)


def kernel(h, edge_index, coord, edge_attr, W_e1, b_e1, W_e2, b_e2,
           W_n1, b_n1, W_n2, b_n2, W_c1, b_c1, W_c2):
    n, d = h.shape
    e = edge_index.shape[1]
    f32 = jnp.float32

    row2d = edge_index[0].reshape(e // _C, _C)
    col2d = edge_index[1].reshape(e // _C, _C)
    coordp = jnp.concatenate(
        [coord, jnp.zeros((n, 13), f32)], axis=1)

    bf16 = jnp.bfloat16
    hr, hc, cr, cc = _sc_gather(h, coordp, row2d, col2d)

    W1h = W_e1[:d].astype(bf16)
    W1c = W_e1[d:2 * d].astype(bf16)
    w1r = W_e1[2 * d:2 * d + 1]
    W1a = W_e1[2 * d + 1:].astype(bf16)
    ef, tc = _tc_edge(hr, hc, edge_attr, cr, cc,
                      W1h, W1c, w1r, W1a, b_e1.reshape(1, -1),
                      W_e2.astype(bf16), b_e2.reshape(1, -1),
                      W_c1.astype(bf16), b_c1.reshape(1, -1),
                      W_c2.reshape(1, -1))

    an0, an1, at0, at1 = _sc_scatter(ef, tc, row2d, n)

    h_out, co = _tc_node(h, coordp, an0, an1, at0, at1,
                         W_n1[:d], W_n1[d:], b_n1.reshape(1, -1),
                         W_n2, b_n2.reshape(1, -1))
    coord_out = co[:, :3].reshape(n, 3, 1)
    return (h_out, coord_out, edge_attr)


# speedup vs baseline: 1.5178x; 1.1131x over previous
import functools

import jax
import jax.numpy as jnp
from jax import lax
from jax.experimental import pallas as pl
from jax.experimental.pallas import tpu as pltpu
from jax.experimental.pallas import tpu_sc as plsc

_C = 128
_G = 8
_BE = 1024
_BN = 1000
_NW = 32
_NT = 16


def _sc_gather(h, xs, ys, zs, row2d, col2d, nchunk):
    n, d = h.shape
    nrow = row2d.shape[0]
    e = nchunk * _C
    ngrp = nrow // _G
    iters = (ngrp + _NW - 1) // _NW
    mesh = plsc.VectorSubcoreMesh(core_axis_name="c", subcore_axis_name="s")
    f32 = jnp.float32

    @functools.partial(
        pl.kernel,
        out_type=(
            jax.ShapeDtypeStruct((e, d), f32),
            jax.ShapeDtypeStruct((e, d), f32),
            jax.ShapeDtypeStruct((nrow, _C), f32),
            jax.ShapeDtypeStruct((nrow, _C), f32),
            jax.ShapeDtypeStruct((nrow, _C), f32),
            jax.ShapeDtypeStruct((nrow, _C), f32),
        ),
        mesh=mesh,
        scratch_types=[
            pltpu.VMEM((_G, _C), jnp.int32),
            pltpu.VMEM((_G, _C), jnp.int32),
            pltpu.VMEM((_C, d), f32),
            pltpu.VMEM((_C, d), f32),
            pltpu.VMEM((_G, _C), f32),
            pltpu.VMEM((_G, _C), f32),
            pltpu.VMEM((_G, _C), f32),
            pltpu.VMEM((_G, _C), f32),
            pltpu.VMEM((n,), f32),
            pltpu.VMEM((n,), f32),
            pltpu.VMEM((n,), f32),
            pltpu.SemaphoreType.DMA,
            pltpu.SemaphoreType.DMA,
        ],
        compiler_params=pltpu.CompilerParams(needs_layout_passes=False),
    )
    def gk(h_hbm, xs_hbm, ys_hbm, zs_hbm, r2_hbm, c2_hbm,
           hr_o, hc_o, dx_o, dy_o, dz_o, rad_o,
           ir8, ic8, bhr, bhc, bdx, bdy, bdz, brad, xt, yt, zt, s1, s2):
        wid = lax.axis_index("s") * 2 + lax.axis_index("c")
        pltpu.sync_copy(xs_hbm, xt)
        pltpu.sync_copy(ys_hbm, yt)
        pltpu.sync_copy(zs_hbm, zt)

        def body(i, carry):
            g = wid + i * _NW

            @pl.when(g < ngrp)
            def _():
                pltpu.sync_copy(r2_hbm.at[pl.ds(g * _G, _G)], ir8)
                pltpu.sync_copy(c2_hbm.at[pl.ds(g * _G, _G)], ic8)
                for j in range(_G):
                    d1 = pltpu.async_copy(h_hbm.at[ir8.at[j]], bhr, s1)
                    d2 = pltpu.async_copy(h_hbm.at[ic8.at[j]], bhc, s2)
                    for q in range(_C // 16):
                        sl = pl.ds(q * 16, 16)
                        vir = ir8[j, sl]
                        vic = ic8[j, sl]
                        dx = (plsc.load_gather(xt, [vir])
                              - plsc.load_gather(xt, [vic]))
                        dy = (plsc.load_gather(yt, [vir])
                              - plsc.load_gather(yt, [vic]))
                        dz = (plsc.load_gather(zt, [vir])
                              - plsc.load_gather(zt, [vic]))
                        bdx[j, sl] = dx
                        bdy[j, sl] = dy
                        bdz[j, sl] = dz
                        brad[j, sl] = dx * dx + dy * dy + dz * dz
                    d1.wait()
                    d2.wait()
                    ch = g * _G + j

                    @pl.when(ch * _C < e)
                    def _():
                        pltpu.sync_copy(bhr, hr_o.at[pl.ds(ch * _C, _C)])
                        pltpu.sync_copy(bhc, hc_o.at[pl.ds(ch * _C, _C)])

                pltpu.sync_copy(bdx, dx_o.at[pl.ds(g * _G, _G)])
                pltpu.sync_copy(bdy, dy_o.at[pl.ds(g * _G, _G)])
                pltpu.sync_copy(bdz, dz_o.at[pl.ds(g * _G, _G)])
                pltpu.sync_copy(brad, rad_o.at[pl.ds(g * _G, _G)])

            return carry

        lax.fori_loop(0, iters, body, 0)

    return gk(h, xs, ys, zs, row2d, col2d)


def _sc_scatter(ef, scalp, dxp, dyp, dzp, row2d, n, nchunk):
    e, d = ef.shape
    nrow = row2d.shape[0]
    ngrp = nrow // _G
    iters = (ngrp + _NT - 1) // _NT
    mesh = plsc.VectorSubcoreMesh(core_axis_name="c", subcore_axis_name="s")
    f32 = jnp.float32
    npt_a = 624
    starts = [s * npt_a for s in range(_NT)]
    lens = [npt_a] * (_NT - 1) + [n - npt_a * (_NT - 1)]
    _ZB = 16

    @functools.partial(
        pl.kernel,
        out_type=(
            jax.ShapeDtypeStruct((n, d), f32),
            jax.ShapeDtypeStruct((n, d), f32),
        ),
        mesh=mesh,
        scratch_types=[
            pltpu.VMEM((_G, _C), jnp.int32),
            pltpu.VMEM((_C, d), f32),
            pltpu.VMEM((_G, _C), f32),
            pltpu.VMEM((_G, _C), f32),
            pltpu.VMEM((_G, _C), f32),
            pltpu.VMEM((_G, _C), f32),
            pltpu.VMEM((_C, d), f32),
            pltpu.VMEM((_ZB, d), f32),
            pltpu.VMEM_SHARED((n, d), f32),
        ],
        compiler_params=pltpu.CompilerParams(needs_layout_passes=False),
    )
    def sk(ef_hbm, sc_hbm, dx_hbm, dy_hbm, dz_hbm, r2_hbm, an_o, at_o,
           ir8, bef, bsc, bdx, bdy, bdz, btc, zb, acc):
        c = lax.axis_index("c")
        s = lax.axis_index("s")
        iot = lax.iota(jnp.int32, 16)

        def zrow(i, carry):
            for j in range(d // 16):
                zb[i, pl.ds(j * 16, 16)] = jnp.zeros((16,), f32)
            return carry

        lax.fori_loop(0, _ZB, zrow, 0)
        for t in range(_NT):
            @pl.when(s == t)
            def _():
                for k in range(lens[t] // _ZB):
                    pltpu.sync_copy(zb, acc.at[pl.ds(starts[t] + k * _ZB, _ZB)])

        def brow(i, carry):
            for j in range(d // 16):
                btc[i, pl.ds(j * 16, 16)] = jnp.zeros((16,), f32)
            btc[i, pl.ds(0, 16)] = (iot == 3).astype(f32)
            return carry

        lax.fori_loop(0, _C, brow, 0)
        plsc.subcore_barrier()

        def body_ef(i, carry):
            g = s + i * _NT

            @pl.when(g < ngrp)
            def _():
                pltpu.sync_copy(r2_hbm.at[pl.ds(g * _G, _G)], ir8)
                for j in range(_G):
                    ch = g * _G + j

                    @pl.when(ch < nchunk)
                    def _():
                        pltpu.sync_copy(ef_hbm.at[pl.ds(ch * _C, _C)], bef)
                        pltpu.sync_copy(bef, acc.at[ir8.at[j]], add=True)

            return carry

        def body_tr(i, carry):
            g = s + i * _NT

            @pl.when(g < ngrp)
            def _():
                pltpu.sync_copy(r2_hbm.at[pl.ds(g * _G, _G)], ir8)
                pltpu.sync_copy(sc_hbm.at[pl.ds(g * _G, _G)], bsc)
                pltpu.sync_copy(dx_hbm.at[pl.ds(g * _G, _G)], bdx)
                pltpu.sync_copy(dy_hbm.at[pl.ds(g * _G, _G)], bdy)
                pltpu.sync_copy(dz_hbm.at[pl.ds(g * _G, _G)], bdz)
                for j in range(_G):
                    ch = g * _G + j

                    @pl.when(ch < nchunk)
                    def _():
                        for q in range(_C // 16):
                            sl = pl.ds(q * 16, 16)
                            vs = bsc[j, sl]
                            rw = q * 16 + iot
                            tx = jnp.clip(bdx[j, sl] * vs, -100.0, 100.0)
                            ty = jnp.clip(bdy[j, sl] * vs, -100.0, 100.0)
                            tz = jnp.clip(bdz[j, sl] * vs, -100.0, 100.0)
                            plsc.store_scatter(
                                btc, [rw, jnp.zeros((16,), jnp.int32)], tx)
                            plsc.store_scatter(
                                btc, [rw, jnp.full((16,), 1, jnp.int32)], ty)
                            plsc.store_scatter(
                                btc, [rw, jnp.full((16,), 2, jnp.int32)], tz)
                        pltpu.sync_copy(btc, acc.at[ir8.at[j]], add=True)

            return carry

        @pl.when(c == 0)
        def _():
            lax.fori_loop(0, iters, body_ef, 0)

        @pl.when(c == 1)
        def _():
            lax.fori_loop(0, iters, body_tr, 0)

        plsc.subcore_barrier()
        for t in range(_NT):
            @pl.when(s == t)
            def _():
                sl = pl.ds(starts[t], lens[t])

                @pl.when(c == 0)
                def _():
                    pltpu.sync_copy(acc.at[sl], an_o.at[sl])

                @pl.when(c == 1)
                def _():
                    pltpu.sync_copy(acc.at[sl], at_o.at[sl])

    return sk(ef, scalp, dxp, dyp, dzp, row2d)


def _tc_edge(hr, hc, ea, radp, W1h, W1c, w1r, W1a, b1, W2, b2, Wc1, bc1, wc2):
    e, d = hr.shape
    he = W2.shape[1]
    nrow = radp.shape[0]
    f32 = jnp.float32
    bf16 = jnp.bfloat16
    grid = (nrow // _G,)

    def body(hr_r, hc_r, ea_r, rad_r, W1h_r, W1c_r, w1r_r, W1a_r, b1_r,
             W2_r, b2_r, Wc1_r, bc1_r, wc2_r, ef_o, sp_o):
        radt = jnp.transpose(rad_r[...])
        xterm = jnp.concatenate(
            [jnp.dot(radt[:, r:r + 1], w1r_r[...], preferred_element_type=f32)
             for r in range(_G)], axis=0)
        x = (jnp.dot(hr_r[...].astype(bf16), W1h_r[...],
                     preferred_element_type=f32)
             + jnp.dot(hc_r[...].astype(bf16), W1c_r[...],
                       preferred_element_type=f32)
             + jnp.dot(ea_r[...].astype(bf16), W1a_r[...],
                       preferred_element_type=f32)
             + xterm + b1_r[...])
        x = jnp.maximum(x, 0.0).astype(bf16)
        ef = jnp.maximum(jnp.dot(x, W2_r[...], preferred_element_type=f32)
                         + b2_r[...], 0.0)
        c1 = jnp.maximum(jnp.dot(ef.astype(bf16), Wc1_r[...],
                                 preferred_element_type=f32)
                         + bc1_r[...], 0.0)
        scal = jnp.sum(c1 * wc2_r[...], axis=1, keepdims=True)
        smat = jnp.concatenate(
            [scal[_C * r:_C * (r + 1), :] for r in range(_G)], axis=1)
        sp_o[...] = jnp.transpose(smat)
        ef_o[...] = ef

    wspec = pl.BlockSpec((d, he), lambda i: (0, 0))
    vspec = pl.BlockSpec((1, he), lambda i: (0, 0))
    return pl.pallas_call(
        body,
        grid=grid,
        in_specs=[
            pl.BlockSpec((_BE, d), lambda i: (i, 0)),
            pl.BlockSpec((_BE, d), lambda i: (i, 0)),
            pl.BlockSpec((_BE, d), lambda i: (i, 0)),
            pl.BlockSpec((_G, _C), lambda i: (i, 0)),
            wspec, wspec, vspec, wspec, vspec,
            wspec, vspec, wspec, vspec, vspec,
        ],
        out_specs=[
            pl.BlockSpec((_BE, he), lambda i: (i, 0)),
            pl.BlockSpec((_G, _C), lambda i: (i, 0)),
        ],
        out_shape=[
            jax.ShapeDtypeStruct((e, he), f32),
            jax.ShapeDtypeStruct((nrow, _C), f32),
        ],
    )(hr, hc, ea, radp, W1h, W1c, w1r, W1a, b1, W2, b2, Wc1, bc1, wc2)


def _tc_node(h, coordp, an, at, Wn1h, Wn1a, bn1, Wn2, bn2):
    n, d = h.shape
    f32 = jnp.float32

    def body(h_r, cp_r, an_r, at_r, Wn1h_r, Wn1a_r, bn1_r, Wn2_r, bn2_r,
             ho_o, co_o):
        hid = jnp.maximum(
            jnp.dot(h_r[...], Wn1h_r[...], preferred_element_type=f32)
            + jnp.dot(an_r[...], Wn1a_r[...], preferred_element_type=f32)
            + bn1_r[...], 0.0)
        ho_o[...] = (jnp.dot(hid, Wn2_r[...], preferred_element_type=f32)
                     + bn2_r[...] + h_r[...])
        aggt = at_r[...][:, :16]
        lane = lax.broadcasted_iota(jnp.int32, (_BN, 16), 1)
        cnt = jnp.sum(jnp.where(lane == 3, aggt, 0.0), axis=1, keepdims=True)
        cnt = jnp.maximum(cnt, 1.0)
        co_o[...] = cp_r[...] + jnp.where(lane < 3, aggt / cnt, 0.0)

    wspec = pl.BlockSpec((d, d), lambda i: (0, 0))
    vspec = pl.BlockSpec((1, d), lambda i: (0, 0))
    return pl.pallas_call(
        body,
        grid=(n // _BN,),
        in_specs=[
            pl.BlockSpec((_BN, d), lambda i: (i, 0)),
            pl.BlockSpec((_BN, 16), lambda i: (i, 0)),
            pl.BlockSpec((_BN, d), lambda i: (i, 0)),
            pl.BlockSpec((_BN, d), lambda i: (i, 0)),
            wspec, wspec, vspec, wspec, vspec,
        ],
        out_specs=[
            pl.BlockSpec((_BN, d), lambda i: (i, 0)),
            pl.BlockSpec((_BN, 16), lambda i: (i, 0)),
        ],
        out_shape=[
            jax.ShapeDtypeStruct((n, d), f32),
            jax.ShapeDtypeStruct((n, 16), f32),
        ],
    )(h, coordp, an, at, Wn1h, Wn1a, bn1, Wn2, bn2)


def kernel(h, edge_index, coord, edge_attr, W_e1, b_e1, W_e2, b_e2,
           W_n1, b_n1, W_n2, b_n2, W_c1, b_c1, W_c2):
    n, d = h.shape
    e = edge_index.shape[1]
    f32 = jnp.float32
    bf16 = jnp.bfloat16

    nchunk = e // _C
    nrow = _G * ((nchunk + _G - 1) // _G)
    pad = nrow * _C - e
    i32 = jnp.int32
    row2d = jnp.concatenate(
        [edge_index[0], jnp.zeros((pad,), i32)]).reshape(nrow, _C)
    col2d = jnp.concatenate(
        [edge_index[1], jnp.zeros((pad,), i32)]).reshape(nrow, _C)
    xs = coord[:, 0]
    ys = coord[:, 1]
    zs = coord[:, 2]
    coordp = jnp.concatenate([coord, jnp.zeros((n, 13), f32)], axis=1)

    hr, hc, dxp, dyp, dzp, radp = _sc_gather(h, xs, ys, zs, row2d, col2d,
                                             nchunk)

    W1h = W_e1[:d].astype(bf16)
    W1c = W_e1[d:2 * d].astype(bf16)
    w1r = W_e1[2 * d:2 * d + 1]
    W1a = W_e1[2 * d + 1:].astype(bf16)
    ef, scalp = _tc_edge(hr, hc, edge_attr, radp,
                         W1h, W1c, w1r, W1a, b_e1.reshape(1, -1),
                         W_e2.astype(bf16), b_e2.reshape(1, -1),
                         W_c1.astype(bf16), b_c1.reshape(1, -1),
                         W_c2.reshape(1, -1))

    an, at = _sc_scatter(ef, scalp, dxp, dyp, dzp, row2d, n, nchunk)

    h_out, co = _tc_node(h, coordp, an, at,
                         W_n1[:d], W_n1[d:], b_n1.reshape(1, -1),
                         W_n2, b_n2.reshape(1, -1))
    coord_out = co[:, :3].reshape(n, 3, 1)
    return (h_out, coord_out, edge_attr)


# speedup vs baseline: 1.5314x; 1.0090x over previous
import functools

import jax
import jax.numpy as jnp
from jax import lax
from jax.experimental import pallas as pl
from jax.experimental.pallas import tpu as pltpu
from jax.experimental.pallas import tpu_sc as plsc

_C = 128
_G = 8
_BE = 1024
_BN = 1000
_NW = 32
_NT = 16


def _sc_gather(h, xs, ys, zs, row2d, col2d, nchunk):
    n, d = h.shape
    nrow = row2d.shape[0]
    e = nchunk * _C
    ngrp = nrow // _G
    iters = (ngrp + _NW - 1) // _NW
    mesh = plsc.VectorSubcoreMesh(core_axis_name="c", subcore_axis_name="s")
    f32 = jnp.float32

    @functools.partial(
        pl.kernel,
        out_type=(
            jax.ShapeDtypeStruct((e, d), f32),
            jax.ShapeDtypeStruct((e, d), f32),
            jax.ShapeDtypeStruct((nrow, _C), f32),
            jax.ShapeDtypeStruct((nrow, _C), f32),
            jax.ShapeDtypeStruct((nrow, _C), f32),
            jax.ShapeDtypeStruct((nrow, _C), f32),
        ),
        mesh=mesh,
        scratch_types=[
            pltpu.VMEM((_G, _C), jnp.int32),
            pltpu.VMEM((_G, _C), jnp.int32),
            pltpu.VMEM((_C, d), f32),
            pltpu.VMEM((_C, d), f32),
            pltpu.VMEM((_G, _C), f32),
            pltpu.VMEM((_G, _C), f32),
            pltpu.VMEM((_G, _C), f32),
            pltpu.VMEM((_G, _C), f32),
            pltpu.VMEM((n,), f32),
            pltpu.VMEM((n,), f32),
            pltpu.VMEM((n,), f32),
            pltpu.SemaphoreType.DMA,
            pltpu.SemaphoreType.DMA,
        ],
        compiler_params=pltpu.CompilerParams(needs_layout_passes=False),
    )
    def gk(h_hbm, xs_hbm, ys_hbm, zs_hbm, r2_hbm, c2_hbm,
           hr_o, hc_o, dx_o, dy_o, dz_o, rad_o,
           ir8, ic8, bhr, bhc, bdx, bdy, bdz, brad, xt, yt, zt, s1, s2):
        wid = lax.axis_index("s") * 2 + lax.axis_index("c")
        pltpu.sync_copy(xs_hbm, xt)
        pltpu.sync_copy(ys_hbm, yt)
        pltpu.sync_copy(zs_hbm, zt)

        def body(i, carry):
            g = wid + i * _NW

            @pl.when(g < ngrp)
            def _():
                pltpu.sync_copy(r2_hbm.at[pl.ds(g * _G, _G)], ir8)
                pltpu.sync_copy(c2_hbm.at[pl.ds(g * _G, _G)], ic8)
                for j in range(_G):
                    d1 = pltpu.async_copy(h_hbm.at[ir8.at[j]], bhr, s1)
                    d2 = pltpu.async_copy(h_hbm.at[ic8.at[j]], bhc, s2)
                    for q in range(_C // 16):
                        sl = pl.ds(q * 16, 16)
                        vir = ir8[j, sl]
                        vic = ic8[j, sl]
                        dx = (plsc.load_gather(xt, [vir])
                              - plsc.load_gather(xt, [vic]))
                        dy = (plsc.load_gather(yt, [vir])
                              - plsc.load_gather(yt, [vic]))
                        dz = (plsc.load_gather(zt, [vir])
                              - plsc.load_gather(zt, [vic]))
                        bdx[j, sl] = dx
                        bdy[j, sl] = dy
                        bdz[j, sl] = dz
                        brad[j, sl] = dx * dx + dy * dy + dz * dz
                    d1.wait()
                    d2.wait()
                    ch = g * _G + j

                    @pl.when(ch * _C < e)
                    def _():
                        pltpu.sync_copy(bhr, hr_o.at[pl.ds(ch * _C, _C)])
                        pltpu.sync_copy(bhc, hc_o.at[pl.ds(ch * _C, _C)])

                pltpu.sync_copy(bdx, dx_o.at[pl.ds(g * _G, _G)])
                pltpu.sync_copy(bdy, dy_o.at[pl.ds(g * _G, _G)])
                pltpu.sync_copy(bdz, dz_o.at[pl.ds(g * _G, _G)])
                pltpu.sync_copy(brad, rad_o.at[pl.ds(g * _G, _G)])

            return carry

        lax.fori_loop(0, iters, body, 0)

    return gk(h, xs, ys, zs, row2d, col2d)


def _sc_scatter(ef, scalp, dxp, dyp, dzp, row2d, n, nchunk):
    e, d = ef.shape
    nrow = row2d.shape[0]
    ngrp = nrow // _G
    iters = (ngrp + _NT - 1) // _NT
    mesh = plsc.VectorSubcoreMesh(core_axis_name="c", subcore_axis_name="s")
    f32 = jnp.float32
    npt_a = 624
    starts = [s * npt_a for s in range(_NT)]
    lens = [npt_a] * (_NT - 1) + [n - npt_a * (_NT - 1)]
    _ZB = 16

    @functools.partial(
        pl.kernel,
        out_type=(
            jax.ShapeDtypeStruct((n, d), f32),
            jax.ShapeDtypeStruct((n, d), f32),
        ),
        mesh=mesh,
        scratch_types=[
            pltpu.VMEM((_G, _C), jnp.int32),
            pltpu.VMEM((_C, d), f32),
            pltpu.VMEM((_G, _C), f32),
            pltpu.VMEM((_G, _C), f32),
            pltpu.VMEM((_G, _C), f32),
            pltpu.VMEM((_G, _C), f32),
            pltpu.VMEM((_C, d), f32),
            pltpu.VMEM((_ZB, d), f32),
            pltpu.VMEM_SHARED((n, d), f32),
        ],
        compiler_params=pltpu.CompilerParams(needs_layout_passes=False),
    )
    def sk(ef_hbm, sc_hbm, dx_hbm, dy_hbm, dz_hbm, r2_hbm, an_o, at_o,
           ir8, bef, bsc, bdx, bdy, bdz, btc, zb, acc):
        c = lax.axis_index("c")
        s = lax.axis_index("s")
        iot = lax.iota(jnp.int32, 16)

        def zrow(i, carry):
            for j in range(d // 16):
                zb[i, pl.ds(j * 16, 16)] = jnp.zeros((16,), f32)
            return carry

        lax.fori_loop(0, _ZB, zrow, 0)
        for t in range(_NT):
            @pl.when(s == t)
            def _():
                for k in range(lens[t] // _ZB):
                    pltpu.sync_copy(zb, acc.at[pl.ds(starts[t] + k * _ZB, _ZB)])

        def brow(i, carry):
            for j in range(d // 16):
                btc[i, pl.ds(j * 16, 16)] = jnp.zeros((16,), f32)
            btc[i, pl.ds(0, 16)] = (iot == 3).astype(f32)
            return carry

        lax.fori_loop(0, _C, brow, 0)
        plsc.subcore_barrier()

        def body_ef(i, carry):
            g = s + i * _NT

            @pl.when(g < ngrp)
            def _():
                pltpu.sync_copy(r2_hbm.at[pl.ds(g * _G, _G)], ir8)
                for j in range(_G):
                    ch = g * _G + j

                    @pl.when(ch < nchunk)
                    def _():
                        pltpu.sync_copy(ef_hbm.at[pl.ds(ch * _C, _C)], bef)
                        pltpu.sync_copy(bef, acc.at[ir8.at[j]], add=True)

            return carry

        def body_tr(i, carry):
            g = s + i * _NT

            @pl.when(g < ngrp)
            def _():
                pltpu.sync_copy(r2_hbm.at[pl.ds(g * _G, _G)], ir8)
                pltpu.sync_copy(sc_hbm.at[pl.ds(g * _G, _G)], bsc)
                pltpu.sync_copy(dx_hbm.at[pl.ds(g * _G, _G)], bdx)
                pltpu.sync_copy(dy_hbm.at[pl.ds(g * _G, _G)], bdy)
                pltpu.sync_copy(dz_hbm.at[pl.ds(g * _G, _G)], bdz)
                for j in range(_G):
                    ch = g * _G + j

                    @pl.when(ch < nchunk)
                    def _():
                        for q in range(_C // 16):
                            sl = pl.ds(q * 16, 16)
                            vs = bsc[j, sl]
                            rw = q * 16 + iot
                            tx = jnp.clip(bdx[j, sl] * vs, -100.0, 100.0)
                            ty = jnp.clip(bdy[j, sl] * vs, -100.0, 100.0)
                            tz = jnp.clip(bdz[j, sl] * vs, -100.0, 100.0)
                            plsc.store_scatter(
                                btc, [rw, jnp.zeros((16,), jnp.int32)], tx)
                            plsc.store_scatter(
                                btc, [rw, jnp.full((16,), 1, jnp.int32)], ty)
                            plsc.store_scatter(
                                btc, [rw, jnp.full((16,), 2, jnp.int32)], tz)
                        pltpu.sync_copy(btc, acc.at[ir8.at[j]], add=True)

            return carry

        @pl.when(c == 0)
        def _():
            lax.fori_loop(0, iters, body_ef, 0)

        @pl.when(c == 1)
        def _():
            lax.fori_loop(0, iters, body_tr, 0)

        plsc.subcore_barrier()
        for t in range(_NT):
            @pl.when(s == t)
            def _():
                sl = pl.ds(starts[t], lens[t])

                @pl.when(c == 0)
                def _():
                    pltpu.sync_copy(acc.at[sl], an_o.at[sl])

                @pl.when(c == 1)
                def _():
                    pltpu.sync_copy(acc.at[sl], at_o.at[sl])

    return sk(ef, scalp, dxp, dyp, dzp, row2d)


def _tc_edge(hr, hc, ea, radp, W1h, W1c, w1r, W1a, b1, W2, b2, Wc1, bc1, wc2):
    e, d = hr.shape
    he = W2.shape[1]
    nrow = radp.shape[0]
    f32 = jnp.float32
    bf16 = jnp.bfloat16
    grid = (nrow // _G,)

    def body(hr_r, hc_r, ea_r, rad_r, W1h_r, W1c_r, w1r_r, W1a_r, b1_r,
             W2_r, b2_r, Wc1_r, bc1_r, wc2_r, ef_o, sp_o):
        gid = lax.broadcasted_iota(jnp.int32, (_BE, _G), 0) // _C
        sel = (lax.broadcasted_iota(jnp.int32, (_BE, _G), 1)
               == gid).astype(bf16)
        lid = lax.broadcasted_iota(jnp.int32, (_BE, _C), 0) % _C
        diag = (lax.broadcasted_iota(jnp.int32, (_BE, _C), 1)
                == lid).astype(f32)
        bcast = jnp.dot(sel, rad_r[...].astype(bf16),
                        preferred_element_type=f32)
        rcol = jnp.sum(bcast * diag, axis=1, keepdims=True)
        x = (jnp.dot(hr_r[...].astype(bf16), W1h_r[...],
                     preferred_element_type=f32)
             + jnp.dot(hc_r[...].astype(bf16), W1c_r[...],
                       preferred_element_type=f32)
             + jnp.dot(ea_r[...].astype(bf16), W1a_r[...],
                       preferred_element_type=f32)
             + rcol * w1r_r[...] + b1_r[...])
        x = jnp.maximum(x, 0.0).astype(bf16)
        ef = jnp.maximum(jnp.dot(x, W2_r[...], preferred_element_type=f32)
                         + b2_r[...], 0.0)
        c1 = jnp.maximum(jnp.dot(ef.astype(bf16), Wc1_r[...],
                                 preferred_element_type=f32)
                         + bc1_r[...], 0.0)
        scal = jnp.sum(c1 * wc2_r[...], axis=1, keepdims=True)
        selt = (lax.broadcasted_iota(jnp.int32, (_G, _BE), 0)
                == lax.broadcasted_iota(jnp.int32, (_G, _BE), 1) // _C
                ).astype(bf16)
        sp_o[...] = jnp.dot(selt, (scal * diag).astype(bf16),
                            preferred_element_type=f32)
        ef_o[...] = ef

    wspec = pl.BlockSpec((d, he), lambda i: (0, 0))
    vspec = pl.BlockSpec((1, he), lambda i: (0, 0))
    return pl.pallas_call(
        body,
        grid=grid,
        in_specs=[
            pl.BlockSpec((_BE, d), lambda i: (i, 0)),
            pl.BlockSpec((_BE, d), lambda i: (i, 0)),
            pl.BlockSpec((_BE, d), lambda i: (i, 0)),
            pl.BlockSpec((_G, _C), lambda i: (i, 0)),
            wspec, wspec, vspec, wspec, vspec,
            wspec, vspec, wspec, vspec, vspec,
        ],
        out_specs=[
            pl.BlockSpec((_BE, he), lambda i: (i, 0)),
            pl.BlockSpec((_G, _C), lambda i: (i, 0)),
        ],
        out_shape=[
            jax.ShapeDtypeStruct((e, he), f32),
            jax.ShapeDtypeStruct((nrow, _C), f32),
        ],
    )(hr, hc, ea, radp, W1h, W1c, w1r, W1a, b1, W2, b2, Wc1, bc1, wc2)


def _tc_node(h, coordp, an, at, Wn1h, Wn1a, bn1, Wn2, bn2):
    n, d = h.shape
    f32 = jnp.float32

    def body(h_r, cp_r, an_r, at_r, Wn1h_r, Wn1a_r, bn1_r, Wn2_r, bn2_r,
             ho_o, co_o):
        hid = jnp.maximum(
            jnp.dot(h_r[...], Wn1h_r[...], preferred_element_type=f32)
            + jnp.dot(an_r[...], Wn1a_r[...], preferred_element_type=f32)
            + bn1_r[...], 0.0)
        ho_o[...] = (jnp.dot(hid, Wn2_r[...], preferred_element_type=f32)
                     + bn2_r[...] + h_r[...])
        aggt = at_r[...][:, :16]
        lane = lax.broadcasted_iota(jnp.int32, (_BN, 16), 1)
        cnt = jnp.sum(jnp.where(lane == 3, aggt, 0.0), axis=1, keepdims=True)
        cnt = jnp.maximum(cnt, 1.0)
        co_o[...] = cp_r[...] + jnp.where(lane < 3, aggt / cnt, 0.0)

    wspec = pl.BlockSpec((d, d), lambda i: (0, 0))
    vspec = pl.BlockSpec((1, d), lambda i: (0, 0))
    return pl.pallas_call(
        body,
        grid=(n // _BN,),
        in_specs=[
            pl.BlockSpec((_BN, d), lambda i: (i, 0)),
            pl.BlockSpec((_BN, 16), lambda i: (i, 0)),
            pl.BlockSpec((_BN, d), lambda i: (i, 0)),
            pl.BlockSpec((_BN, d), lambda i: (i, 0)),
            wspec, wspec, vspec, wspec, vspec,
        ],
        out_specs=[
            pl.BlockSpec((_BN, d), lambda i: (i, 0)),
            pl.BlockSpec((_BN, 16), lambda i: (i, 0)),
        ],
        out_shape=[
            jax.ShapeDtypeStruct((n, d), f32),
            jax.ShapeDtypeStruct((n, 16), f32),
        ],
    )(h, coordp, an, at, Wn1h, Wn1a, bn1, Wn2, bn2)


def kernel(h, edge_index, coord, edge_attr, W_e1, b_e1, W_e2, b_e2,
           W_n1, b_n1, W_n2, b_n2, W_c1, b_c1, W_c2):
    n, d = h.shape
    e = edge_index.shape[1]
    f32 = jnp.float32
    bf16 = jnp.bfloat16

    nchunk = e // _C
    nrow = _G * ((nchunk + _G - 1) // _G)
    pad = nrow * _C - e
    i32 = jnp.int32
    row2d = jnp.concatenate(
        [edge_index[0], jnp.zeros((pad,), i32)]).reshape(nrow, _C)
    col2d = jnp.concatenate(
        [edge_index[1], jnp.zeros((pad,), i32)]).reshape(nrow, _C)
    xs = coord[:, 0]
    ys = coord[:, 1]
    zs = coord[:, 2]
    coordp = jnp.concatenate([coord, jnp.zeros((n, 13), f32)], axis=1)

    hr, hc, dxp, dyp, dzp, radp = _sc_gather(h, xs, ys, zs, row2d, col2d,
                                             nchunk)

    W1h = W_e1[:d].astype(bf16)
    W1c = W_e1[d:2 * d].astype(bf16)
    w1r = W_e1[2 * d:2 * d + 1]
    W1a = W_e1[2 * d + 1:].astype(bf16)
    ef, scalp = _tc_edge(hr, hc, edge_attr, radp,
                         W1h, W1c, w1r, W1a, b_e1.reshape(1, -1),
                         W_e2.astype(bf16), b_e2.reshape(1, -1),
                         W_c1.astype(bf16), b_c1.reshape(1, -1),
                         W_c2.reshape(1, -1))

    an, at = _sc_scatter(ef, scalp, dxp, dyp, dzp, row2d, n, nchunk)

    h_out, co = _tc_node(h, coordp, an, at,
                         W_n1[:d], W_n1[d:], b_n1.reshape(1, -1),
                         W_n2, b_n2.reshape(1, -1))
    coord_out = co[:, :3].reshape(n, 3, 1)
    return (h_out, coord_out, edge_attr)
